# baseline (speedup 1.0000x reference)
import jax
import jax.numpy as jnp
from jax import lax
from jax.experimental import pallas as pl
from jax.experimental.pallas import tpu as pltpu
from jax.experimental.pallas import tpu_sc as plsc

N_NODES = 10000
D = 128
B = 128
NCORES = 2
NSUB = 16
NW = NCORES * NSUB
NPAD = 10240
RPS = NPAD // NSUB
PAD_ROW = NPAD - 8
NBUF = 2
G = 8
C0 = 160
C1 = 0
TOTC = NSUB * (C0 + C1)

_mesh = plsc.VectorSubcoreMesh(core_axis_name="c", subcore_axis_name="s")


def _hist_body(dst_hbm, out_hbm, deg_acc, ones_v, idx_v, zero_v):
    c = lax.axis_index("c")
    s = lax.axis_index("s")

    def fill_zero(i, carry):
        zero_v[pl.ds(i * 16, 16)] = jnp.zeros((16,), jnp.float32)
        return carry

    lax.fori_loop(0, RPS // 16, fill_zero, 0)

    def fill_one(i, carry):
        ones_v[pl.ds(i * 16, 16)] = jnp.ones((16,), jnp.float32)
        return carry

    lax.fori_loop(0, B // 16, fill_one, 0)

    pltpu.sync_copy(zero_v, deg_acc.at[pl.ds(s * RPS, RPS)])
    plsc.subcore_barrier()

    def run(base, nchunks):
        def grp(g, carry):
            pltpu.sync_copy(dst_hbm.at[pl.ds(base + g * G, G)], idx_v)

            def body(j, carry2):
                pltpu.sync_copy(ones_v, deg_acc.at[idx_v.at[j]], add=True)
                return carry2

            lax.fori_loop(0, G, body, 0)
            return carry

        lax.fori_loop(0, nchunks // G, grp, 0)

    @pl.when(c == 0)
    def _c0():
        run(s * C0, C0)

    @pl.when(c == 1)
    def _c1():
        run(NSUB * C0 + s * C1, C1)

    plsc.subcore_barrier()
    pltpu.sync_copy(deg_acc.at[pl.ds(s * RPS, RPS)],
                    out_hbm.at[c, pl.ds(s * RPS, RPS)])


_hist = pl.kernel(
    _hist_body,
    out_type=jax.ShapeDtypeStruct((NCORES, NPAD), jnp.float32),
    mesh=_mesh,
    scratch_types=[
        pltpu.VMEM_SHARED((NPAD,), jnp.float32),
        pltpu.VMEM((B,), jnp.float32),
        pltpu.VMEM((G, B), jnp.int32),
        pltpu.VMEM((RPS,), jnp.float32),
    ],
)


def _scatter_body(table_hbm, src_hbm, dst_hbm, out_hbm,
                  acc, rows_v, sidx_v, didx_v, gsem):
    c = lax.axis_index("c")
    s = lax.axis_index("s")

    def zero_row(r, carry):
        for k in range(D // 16):
            rows_v[0, r, pl.ds(k * 16, 16)] = jnp.zeros((16,), jnp.float32)
        return carry

    lax.fori_loop(0, B, zero_row, 0)
    for k in range(RPS // B):
        pltpu.sync_copy(rows_v.at[0], acc.at[pl.ds(s * RPS + k * B, B)])
    plsc.subcore_barrier()

    def run(base, nchunks):
        def grp(g, carry):
            pltpu.sync_copy(src_hbm.at[pl.ds(base + g * G, G)], sidx_v)
            pltpu.sync_copy(dst_hbm.at[pl.ds(base + g * G, G)], didx_v)
            for b in range(NBUF):
                pltpu.make_async_copy(table_hbm.at[sidx_v.at[b]],
                                      rows_v.at[b], gsem.at[b]).start()

            def body(j, carry2):
                b = lax.rem(j, NBUF)
                pltpu.make_async_copy(table_hbm.at[sidx_v.at[j]],
                                      rows_v.at[b], gsem.at[b]).wait()
                pltpu.sync_copy(rows_v.at[b], acc.at[didx_v.at[j]], add=True)

                @pl.when(j + NBUF < G)
                def _issue():
                    pltpu.make_async_copy(table_hbm.at[sidx_v.at[j + NBUF]],
                                          rows_v.at[b], gsem.at[b]).start()

                return carry2

            lax.fori_loop(0, G, body, 0)
            return carry

        lax.fori_loop(0, nchunks // G, grp, 0)

    @pl.when(c == 0)
    def _c0():
        run(s * C0, C0)

    @pl.when(c == 1)
    def _c1():
        run(NSUB * C0 + s * C1, C1)

    plsc.subcore_barrier()
    pltpu.sync_copy(acc.at[pl.ds(s * RPS, RPS)],
                    out_hbm.at[c, pl.ds(s * RPS, RPS)])


_scatter = pl.kernel(
    _scatter_body,
    out_type=jax.ShapeDtypeStruct((NCORES, NPAD, D), jnp.float32),
    mesh=_mesh,
    scratch_types=[
        pltpu.VMEM_SHARED((NPAD, D), jnp.float32),
        pltpu.VMEM((NBUF, B, D), jnp.float32),
        pltpu.VMEM((G, B), jnp.int32),
        pltpu.VMEM((G, B), jnp.int32),
        pltpu.SemaphoreType.DMA((NBUF,)),
    ],
)


def _dinv_body(p_ref, o_ref):
    o_ref[...] = lax.rsqrt(p_ref[0, :] + p_ref[1, :] + 1.0)


def _mm1_body(x_ref, w_ref, dv_ref, o_ref):
    h = jnp.dot(x_ref[...], w_ref[...], preferred_element_type=jnp.float32)
    o_ref[...] = h * dv_ref[...][:, None]


def _layer2_body(a0_ref, a1_ref, hp_ref, dv_ref, b_ref, w_ref, o_ref):
    dv = dv_ref[...][:, None]
    pre = (a0_ref[...] + a1_ref[...] + hp_ref[...]) * dv + b_ref[...][None, :]
    g = pre * 0.5 * (1.0 + lax.erf(pre * 0.7071067811865476))
    o_ref[...] = jnp.dot(g, w_ref[...],
                         preferred_element_type=jnp.float32) * dv


def _final_body(a0_ref, a1_ref, hp_ref, dv_ref, b_ref, o_ref):
    o_ref[...] = ((a0_ref[...] + a1_ref[...] + hp_ref[...])
                  * dv_ref[...][:, None] + b_ref[...][None, :])


RB = 1024
_row_spec = pl.BlockSpec((RB, D), lambda i: (i, 0))
_dv_spec = pl.BlockSpec((RB,), lambda i: (i,))
_w_spec = pl.BlockSpec((D, D), lambda i: (0, 0))
_b_spec = pl.BlockSpec((D,), lambda i: (0,))
_mat_out = jax.ShapeDtypeStruct((NPAD, D), jnp.float32)
_grid = (NPAD // RB,)

_dinv_call = pl.pallas_call(
    _dinv_body, out_shape=jax.ShapeDtypeStruct((NPAD,), jnp.float32))

_mm1_call = pl.pallas_call(
    _mm1_body, grid=_grid,
    in_specs=[_row_spec, _w_spec, _dv_spec],
    out_specs=_row_spec, out_shape=_mat_out)

_layer2_call = pl.pallas_call(
    _layer2_body, grid=_grid,
    in_specs=[_row_spec, _row_spec, _row_spec, _dv_spec, _b_spec, _w_spec],
    out_specs=_row_spec, out_shape=_mat_out)

_final_call = pl.pallas_call(
    _final_body, grid=_grid,
    in_specs=[_row_spec, _row_spec, _row_spec, _dv_spec, _b_spec],
    out_specs=_row_spec, out_shape=_mat_out)


@jax.jit
def kernel(x, edge_index, W1, b1, W2, b2):
    n_edges = edge_index.shape[1]
    epad = TOTC * B
    assert epad >= n_edges

    src = edge_index[0].astype(jnp.int32)
    dst = edge_index[1].astype(jnp.int32)
    src2 = jnp.concatenate(
        [src, jnp.zeros((epad - n_edges,), jnp.int32)]).reshape(TOTC, B)
    dst2 = jnp.concatenate(
        [dst, jnp.full((epad - n_edges,), PAD_ROW, jnp.int32)]
    ).reshape(TOTC, B)
    x_p = jnp.zeros((NPAD, D), jnp.float32).at[:x.shape[0]].set(x)

    p = _hist(dst2)
    dinv = _dinv_call(p)
    h1p = _mm1_call(x_p, W1, dinv)
    A1 = _scatter(h1p, src2, dst2)
    h2p = _layer2_call(A1[0], A1[1], h1p, dinv, b1, W2)
    A2 = _scatter(h2p, src2, dst2)
    outp = _final_call(A2[0], A2[1], h2p, dinv, b2)
    return outp[:x.shape[0]]


# speedup vs baseline: 1.2560x; 1.2560x over previous
import jax
import jax.numpy as jnp
from jax import lax
from jax.experimental import pallas as pl
from jax.experimental.pallas import tpu as pltpu
from jax.experimental.pallas import tpu_sc as plsc

N_NODES = 10000
D = 128
B = 128
NCORES = 2
NSUB = 16
NW = NCORES * NSUB
NPAD = 10240
RPS = NPAD // NSUB
PAD_ROW = NPAD - 8
NBUF = 2
G = 8
C0 = 80
C1 = 80
TOTC = NSUB * (C0 + C1)

_mesh = plsc.VectorSubcoreMesh(core_axis_name="c", subcore_axis_name="s")


def _hist_body(dst_hbm, out_hbm, deg_acc, ones_v, idx_v, zero_v):
    c = lax.axis_index("c")
    s = lax.axis_index("s")

    def fill_zero(i, carry):
        zero_v[pl.ds(i * 16, 16)] = jnp.zeros((16,), jnp.float32)
        return carry

    lax.fori_loop(0, RPS // 16, fill_zero, 0)

    def fill_one(i, carry):
        ones_v[pl.ds(i * 16, 16)] = jnp.ones((16,), jnp.float32)
        return carry

    lax.fori_loop(0, B // 16, fill_one, 0)

    pltpu.sync_copy(zero_v, deg_acc.at[pl.ds(s * RPS, RPS)])
    plsc.subcore_barrier()

    def run(base, nchunks):
        def grp(g, carry):
            pltpu.sync_copy(dst_hbm.at[pl.ds(base + g * G, G)], idx_v)

            def body(j, carry2):
                pltpu.sync_copy(ones_v, deg_acc.at[idx_v.at[j]], add=True)
                return carry2

            lax.fori_loop(0, G, body, 0)
            return carry

        lax.fori_loop(0, nchunks // G, grp, 0)

    @pl.when(c == 0)
    def _c0():
        run(s * C0, C0)

    @pl.when(c == 1)
    def _c1():
        run(NSUB * C0 + s * C1, C1)

    plsc.subcore_barrier()
    pltpu.sync_copy(deg_acc.at[pl.ds(s * RPS, RPS)],
                    out_hbm.at[c, pl.ds(s * RPS, RPS)])


_hist = pl.kernel(
    _hist_body,
    out_type=jax.ShapeDtypeStruct((NCORES, NPAD), jnp.float32),
    mesh=_mesh,
    scratch_types=[
        pltpu.VMEM_SHARED((NPAD,), jnp.float32),
        pltpu.VMEM((B,), jnp.float32),
        pltpu.VMEM((G, B), jnp.int32),
        pltpu.VMEM((RPS,), jnp.float32),
    ],
)


def _scatter_body(table_hbm, src_hbm, dst_hbm, out_hbm,
                  acc, rows_v, sidx_v, didx_v, gsem):
    c = lax.axis_index("c")
    s = lax.axis_index("s")

    def zero_row(r, carry):
        for k in range(D // 16):
            rows_v[0, r, pl.ds(k * 16, 16)] = jnp.zeros((16,), jnp.float32)
        return carry

    lax.fori_loop(0, B, zero_row, 0)
    for k in range(RPS // B):
        pltpu.sync_copy(rows_v.at[0], acc.at[pl.ds(s * RPS + k * B, B)])
    plsc.subcore_barrier()

    def run(base, nchunks):
        def grp(g, carry):
            pltpu.sync_copy(src_hbm.at[pl.ds(base + g * G, G)], sidx_v)
            pltpu.sync_copy(dst_hbm.at[pl.ds(base + g * G, G)], didx_v)
            for b in range(NBUF):
                pltpu.make_async_copy(table_hbm.at[sidx_v.at[b]],
                                      rows_v.at[b], gsem.at[b]).start()

            def body(j, carry2):
                b = lax.rem(j, NBUF)
                pltpu.make_async_copy(table_hbm.at[sidx_v.at[j]],
                                      rows_v.at[b], gsem.at[b]).wait()
                pltpu.sync_copy(rows_v.at[b], acc.at[didx_v.at[j]], add=True)

                @pl.when(j + NBUF < G)
                def _issue():
                    pltpu.make_async_copy(table_hbm.at[sidx_v.at[j + NBUF]],
                                          rows_v.at[b], gsem.at[b]).start()

                return carry2

            lax.fori_loop(0, G, body, 0)
            return carry

        lax.fori_loop(0, nchunks // G, grp, 0)

    @pl.when(c == 0)
    def _c0():
        with jax.named_scope("edges_core0"):
            run(s * C0, C0)

    @pl.when(c == 1)
    def _c1():
        with jax.named_scope("edges_core1"):
            run(NSUB * C0 + s * C1, C1)

    plsc.subcore_barrier()
    with jax.named_scope("epilogue"):
        pltpu.sync_copy(acc.at[pl.ds(s * RPS, RPS)],
                        out_hbm.at[c, pl.ds(s * RPS, RPS)])


_scatter = pl.kernel(
    _scatter_body,
    out_type=jax.ShapeDtypeStruct((NCORES, NPAD, D), jnp.float32),
    mesh=_mesh,
    scratch_types=[
        pltpu.VMEM_SHARED((NPAD, D), jnp.float32),
        pltpu.VMEM((NBUF, B, D), jnp.float32),
        pltpu.VMEM((G, B), jnp.int32),
        pltpu.VMEM((G, B), jnp.int32),
        pltpu.SemaphoreType.DMA((NBUF,)),
    ],
)


def _dinv_body(p_ref, o_ref):
    o_ref[...] = lax.rsqrt(p_ref[0, :] + p_ref[1, :] + 1.0)


def _mm1_body(x_ref, w_ref, dv_ref, o_ref):
    h = jnp.dot(x_ref[...], w_ref[...], preferred_element_type=jnp.float32)
    o_ref[...] = h * dv_ref[...][:, None]


def _layer2_body(a0_ref, a1_ref, hp_ref, dv_ref, b_ref, w_ref, o_ref):
    dv = dv_ref[...][:, None]
    pre = (a0_ref[...] + a1_ref[...] + hp_ref[...]) * dv + b_ref[...][None, :]
    g = pre * 0.5 * (1.0 + lax.erf(pre * 0.7071067811865476))
    o_ref[...] = jnp.dot(g, w_ref[...],
                         preferred_element_type=jnp.float32) * dv


def _final_body(a0_ref, a1_ref, hp_ref, dv_ref, b_ref, o_ref):
    o_ref[...] = ((a0_ref[...] + a1_ref[...] + hp_ref[...])
                  * dv_ref[...][:, None] + b_ref[...][None, :])


RB = 1024
_row_spec = pl.BlockSpec((RB, D), lambda i: (i, 0))
_dv_spec = pl.BlockSpec((RB,), lambda i: (i,))
_w_spec = pl.BlockSpec((D, D), lambda i: (0, 0))
_b_spec = pl.BlockSpec((D,), lambda i: (0,))
_mat_out = jax.ShapeDtypeStruct((NPAD, D), jnp.float32)
_grid = (NPAD // RB,)

_dinv_call = pl.pallas_call(
    _dinv_body, out_shape=jax.ShapeDtypeStruct((NPAD,), jnp.float32))

_mm1_call = pl.pallas_call(
    _mm1_body, grid=_grid,
    in_specs=[_row_spec, _w_spec, _dv_spec],
    out_specs=_row_spec, out_shape=_mat_out)

_layer2_call = pl.pallas_call(
    _layer2_body, grid=_grid,
    in_specs=[_row_spec, _row_spec, _row_spec, _dv_spec, _b_spec, _w_spec],
    out_specs=_row_spec, out_shape=_mat_out)

_final_call = pl.pallas_call(
    _final_body, grid=_grid,
    in_specs=[_row_spec, _row_spec, _row_spec, _dv_spec, _b_spec],
    out_specs=_row_spec, out_shape=_mat_out)


@jax.jit
def kernel(x, edge_index, W1, b1, W2, b2):
    n_edges = edge_index.shape[1]
    epad = TOTC * B
    assert epad >= n_edges

    src = edge_index[0].astype(jnp.int32)
    dst = edge_index[1].astype(jnp.int32)
    src2 = jnp.concatenate(
        [src, jnp.zeros((epad - n_edges,), jnp.int32)]).reshape(TOTC, B)
    dst2 = jnp.concatenate(
        [dst, jnp.full((epad - n_edges,), PAD_ROW, jnp.int32)]
    ).reshape(TOTC, B)
    x_p = jnp.zeros((NPAD, D), jnp.float32).at[:x.shape[0]].set(x)

    p = _hist(dst2)
    dinv = _dinv_call(p)
    h1p = _mm1_call(x_p, W1, dinv)
    A1 = _scatter(h1p, src2, dst2)
    h2p = _layer2_call(A1[0], A1[1], h1p, dinv, b1, W2)
    A2 = _scatter(h2p, src2, dst2)
    outp = _final_call(A2[0], A2[1], h2p, dinv, b2)
    return outp[:x.shape[0]]


# speedup vs baseline: 3.2828x; 2.6137x over previous
import jax
import jax.numpy as jnp
from jax import lax
from jax.experimental import pallas as pl
from jax.experimental.pallas import tpu as pltpu
from jax.experimental.pallas import tpu_sc as plsc

N_NODES = 10000
D = 128
B = 128
NCORES = 2
NSUB = 16
NW = NCORES * NSUB
NPAD = 10240
RPS = NPAD // NSUB
PAD_ROW = NPAD - 8
NBUF = 2
G = 8
C0 = 80
C1 = 80
TOTC = NSUB * (C0 + C1)

_mesh = plsc.VectorSubcoreMesh(core_axis_name="c", subcore_axis_name="s")


def _hist_body(dst_hbm, out_hbm, deg_acc, ones_v, idx_v, zero_v):
    c = lax.axis_index("c")
    s = lax.axis_index("s")

    def fill_zero(i, carry):
        zero_v[pl.ds(i * 16, 16)] = jnp.zeros((16,), jnp.float32)
        return carry

    lax.fori_loop(0, RPS // 16, fill_zero, 0)

    def fill_one(i, carry):
        ones_v[pl.ds(i * 16, 16)] = jnp.ones((16,), jnp.float32)
        return carry

    lax.fori_loop(0, B // 16, fill_one, 0)

    pltpu.sync_copy(zero_v, deg_acc.at[pl.ds(s * RPS, RPS)])
    plsc.subcore_barrier()

    def run(base, nchunks):
        def grp(g, carry):
            pltpu.sync_copy(dst_hbm.at[pl.ds(base + g * G, G)], idx_v)

            def body(j, carry2):
                pltpu.sync_copy(ones_v, deg_acc.at[idx_v.at[j]], add=True)
                return carry2

            lax.fori_loop(0, G, body, 0)
            return carry

        lax.fori_loop(0, nchunks // G, grp, 0)

    @pl.when(c == 0)
    def _c0():
        run(s * C0, C0)

    @pl.when(c == 1)
    def _c1():
        run(NSUB * C0 + s * C1, C1)

    plsc.subcore_barrier()
    pltpu.sync_copy(deg_acc.at[pl.ds(s * RPS, RPS)],
                    out_hbm.at[c, pl.ds(s * RPS, RPS)])


_hist = pl.kernel(
    _hist_body,
    out_type=jax.ShapeDtypeStruct((NCORES, NPAD), jnp.float32),
    mesh=_mesh,
    scratch_types=[
        pltpu.VMEM_SHARED((NPAD,), jnp.float32),
        pltpu.VMEM((B,), jnp.float32),
        pltpu.VMEM((G, B), jnp.int32),
        pltpu.VMEM((RPS,), jnp.float32),
    ],
)


def _scatter_body(table_hbm, src_hbm, dst_hbm, out_hbm,
                  acc, rows_v, sidx_v, didx_v, gsem):
    c = lax.axis_index("c")
    s = lax.axis_index("s")

    def zero_row(r, carry):
        for k in range(D // 16):
            rows_v[0, r, pl.ds(k * 16, 16)] = jnp.zeros((16,), jnp.float32)
        return carry

    lax.fori_loop(0, B, zero_row, 0)
    for k in range(RPS // B):
        pltpu.sync_copy(rows_v.at[0], acc.at[pl.ds(s * RPS + k * B, B)])
    plsc.subcore_barrier()

    def run(base, nchunks):
        def grp(g, carry):
            pltpu.sync_copy(src_hbm.at[pl.ds(base + g * G, G)], sidx_v)
            pltpu.sync_copy(dst_hbm.at[pl.ds(base + g * G, G)], didx_v)
            for b in range(NBUF):
                pltpu.make_async_copy(table_hbm.at[sidx_v.at[b]],
                                      rows_v.at[b], gsem.at[b]).start()

            def body(j, carry2):
                b = lax.rem(j, NBUF)
                pltpu.make_async_copy(table_hbm.at[sidx_v.at[j]],
                                      rows_v.at[b], gsem.at[b]).wait()
                pltpu.sync_copy(rows_v.at[b], acc.at[didx_v.at[j]], add=True)

                @pl.when(j + NBUF < G)
                def _issue():
                    pltpu.make_async_copy(table_hbm.at[sidx_v.at[j + NBUF]],
                                          rows_v.at[b], gsem.at[b]).start()

                return carry2

            lax.fori_loop(0, G, body, 0)
            return carry

        lax.fori_loop(0, nchunks // G, grp, 0)

    @pl.when(c == 0)
    def _c0():
        with jax.named_scope("edges_core0"):
            run(s * C0, C0)

    @pl.when(c == 1)
    def _c1():
        with jax.named_scope("edges_core1"):
            run(NSUB * C0 + s * C1, C1)

    plsc.subcore_barrier()
    with jax.named_scope("epilogue"):
        pltpu.sync_copy(acc.at[pl.ds(s * RPS, RPS)],
                        out_hbm.at[c, pl.ds(s * RPS, RPS)])


_scatter = pl.kernel(
    _scatter_body,
    out_type=jax.ShapeDtypeStruct((NCORES, NPAD, D), jnp.float32),
    mesh=_mesh,
    scratch_types=[
        pltpu.VMEM_SHARED((NPAD, D), jnp.float32),
        pltpu.VMEM((NBUF, B, D), jnp.float32),
        pltpu.VMEM((G, B), jnp.int32),
        pltpu.VMEM((G, B), jnp.int32),
        pltpu.SemaphoreType.DMA((NBUF,)),
    ],
)


def _dinv_body(p_ref, o_ref):
    o_ref[...] = lax.rsqrt(p_ref[0, :] + p_ref[1, :] + 1.0)


def _mm1_body(x_ref, w_ref, dv_ref, o_ref):
    h = jnp.dot(x_ref[...], w_ref[...], preferred_element_type=jnp.float32)
    o_ref[...] = h * dv_ref[...][:, None]


def _layer2_body(a0_ref, a1_ref, hp_ref, dv_ref, b_ref, w_ref, o_ref):
    dv = dv_ref[...][:, None]
    pre = (a0_ref[...] + a1_ref[...] + hp_ref[...]) * dv + b_ref[...][None, :]
    g = pre * 0.5 * (1.0 + lax.erf(pre * 0.7071067811865476))
    o_ref[...] = jnp.dot(g, w_ref[...],
                         preferred_element_type=jnp.float32) * dv


def _final_body(a0_ref, a1_ref, hp_ref, dv_ref, b_ref, o_ref):
    o_ref[...] = ((a0_ref[...] + a1_ref[...] + hp_ref[...])
                  * dv_ref[...][:, None] + b_ref[...][None, :])


RB = 1024
_row_spec = pl.BlockSpec((RB, D), lambda i: (i, 0))
_dv_spec = pl.BlockSpec((RB,), lambda i: (i,))
_w_spec = pl.BlockSpec((D, D), lambda i: (0, 0))
_b_spec = pl.BlockSpec((D,), lambda i: (0,))
_mat_out = jax.ShapeDtypeStruct((NPAD, D), jnp.float32)
_grid = (NPAD // RB,)

_dinv_call = pl.pallas_call(
    _dinv_body, out_shape=jax.ShapeDtypeStruct((NPAD,), jnp.float32))

_mm1_call = pl.pallas_call(
    _mm1_body, grid=_grid,
    in_specs=[_row_spec, _w_spec, _dv_spec],
    out_specs=_row_spec, out_shape=_mat_out)

_layer2_call = pl.pallas_call(
    _layer2_body, grid=_grid,
    in_specs=[_row_spec, _row_spec, _row_spec, _dv_spec, _b_spec, _w_spec],
    out_specs=_row_spec, out_shape=_mat_out)

_final_call = pl.pallas_call(
    _final_body, grid=_grid,
    in_specs=[_row_spec, _row_spec, _row_spec, _dv_spec, _b_spec],
    out_specs=_row_spec, out_shape=_mat_out)


@jax.jit
def kernel(x, edge_index, W1, b1, W2, b2):
    n_edges = edge_index.shape[1]
    epad = TOTC * B
    assert epad >= n_edges

    src = edge_index[0].astype(jnp.int32)
    dst = edge_index[1].astype(jnp.int32)
    npad_e = epad - n_edges
    pad_i = jnp.arange(npad_e, dtype=jnp.int32)
    src2 = jnp.concatenate([src, pad_i % 240]).reshape(TOTC, B)
    dst2 = jnp.concatenate(
        [dst, N_NODES + pad_i % (NPAD - N_NODES)]).reshape(TOTC, B)
    x_p = jnp.zeros((NPAD, D), jnp.float32).at[:x.shape[0]].set(x)

    p = _hist(dst2)
    dinv = _dinv_call(p)
    h1p = _mm1_call(x_p, W1, dinv)
    A1 = _scatter(h1p, src2, dst2)
    h2p = _layer2_call(A1[0], A1[1], h1p, dinv, b1, W2)
    A2 = _scatter(h2p, src2, dst2)
    outp = _final_call(A2[0], A2[1], h2p, dinv, b2)
    return outp[:x.shape[0]]


# speedup vs baseline: 3.3558x; 1.0222x over previous
import jax
import jax.numpy as jnp
from jax import lax
from jax.experimental import pallas as pl
from jax.experimental.pallas import tpu as pltpu
from jax.experimental.pallas import tpu_sc as plsc

N_NODES = 10000
D = 128
B = 128
CHUNKS = 2500
NCORES = 2
NSUB = 16
NW = NCORES * NSUB
CPW = CHUNKS // NW
XTRA = CHUNKS - NW * CPW
NROW = 10016
RPS = NROW // NSUB
NHIST = 10240
RPSH = NHIST // NSUB
NBUF = 3
G = 6
FG = CPW // G
NPAD = 10240

_mesh = plsc.VectorSubcoreMesh(core_axis_name="c", subcore_axis_name="s")
_sc_params = pltpu.CompilerParams(use_tc_tiling_on_sc=False)


def _worker_base(c, s):
    wid = c * NSUB + s
    return wid, CPW * wid + jnp.minimum(wid, XTRA)


def _hist_body(dst_hbm, out_hbm, deg_acc, ones_v, idx_v, zero_v):
    c = lax.axis_index("c")
    s = lax.axis_index("s")
    wid, base = _worker_base(c, s)

    def fill_zero(i, carry):
        zero_v[pl.ds(i * 16, 16)] = jnp.zeros((16,), jnp.float32)
        return carry

    lax.fori_loop(0, RPSH // 16, fill_zero, 0)

    def fill_one(i, carry):
        ones_v[pl.ds(i * 16, 16)] = jnp.ones((16,), jnp.float32)
        return carry

    lax.fori_loop(0, B // 16, fill_one, 0)

    pltpu.sync_copy(zero_v, deg_acc.at[pl.ds(s * RPSH, RPSH)])
    plsc.subcore_barrier()

    def body(j, carry2):
        pltpu.sync_copy(ones_v, deg_acc.at[idx_v.at[j]], add=True)
        return carry2

    def grp(g, carry):
        pltpu.sync_copy(dst_hbm.at[pl.ds(base + g * G, G)], idx_v)
        lax.fori_loop(0, G, body, 0)
        return carry

    lax.fori_loop(0, FG, grp, 0)

    @pl.when(wid < XTRA)
    def _tail():
        pltpu.sync_copy(dst_hbm.at[pl.ds(base + FG * G, 1)],
                        idx_v.at[pl.ds(0, 1)])
        pltpu.sync_copy(ones_v, deg_acc.at[idx_v.at[0]], add=True)

    plsc.subcore_barrier()
    pltpu.sync_copy(deg_acc.at[pl.ds(s * RPSH, RPSH)],
                    out_hbm.at[c, pl.ds(s * RPSH, RPSH)])


_hist = pl.kernel(
    _hist_body,
    out_type=jax.ShapeDtypeStruct((NCORES, NHIST), jnp.float32),
    mesh=_mesh,
    scratch_types=[
        pltpu.VMEM_SHARED((NHIST,), jnp.float32),
        pltpu.VMEM((B,), jnp.float32),
        pltpu.VMEM((G, B), jnp.int32),
        pltpu.VMEM((RPSH,), jnp.float32),
    ],
    compiler_params=_sc_params,
)


def _scatter_body(table_hbm, src_hbm, dst_hbm, out_hbm,
                  acc, rows_v, sidx_v, didx_v, gsem, ssem):
    c = lax.axis_index("c")
    s = lax.axis_index("s")
    wid, base = _worker_base(c, s)

    def zero_row(r, carry):
        for k in range(D // 16):
            rows_v[0, r, pl.ds(k * 16, 16)] = jnp.zeros((16,), jnp.float32)
        return carry

    lax.fori_loop(0, B, zero_row, 0)
    for k in range(RPS // B):
        pltpu.sync_copy(rows_v.at[0], acc.at[pl.ds(s * RPS + k * B, B)])
    _rem = RPS % B
    pltpu.sync_copy(rows_v.at[0, pl.ds(0, _rem)],
                    acc.at[pl.ds(s * RPS + (RPS // B) * B, _rem)])
    plsc.subcore_barrier()

    def emit_group(gbase, n):
        pltpu.sync_copy(src_hbm.at[pl.ds(gbase, n)], sidx_v.at[pl.ds(0, n)])
        pltpu.sync_copy(dst_hbm.at[pl.ds(gbase, n)], didx_v.at[pl.ds(0, n)])
        for j in range(min(2, n)):
            pltpu.make_async_copy(table_hbm.at[sidx_v.at[j]],
                                  rows_v.at[j], gsem.at[j]).start()

        def body(j, carry):
            b = lax.rem(j, NBUF)
            pltpu.make_async_copy(table_hbm.at[sidx_v.at[j]],
                                  rows_v.at[b], gsem.at[b]).wait()
            pltpu.make_async_copy(rows_v.at[b], acc.at[didx_v.at[j]],
                                  ssem.at[b]).start(add=True)

            @pl.when(j + 2 < n)
            def _issue():
                bp = lax.rem(j + 2, NBUF)

                @pl.when(j >= 1)
                def _wait_prev():
                    pltpu.make_async_copy(rows_v.at[bp],
                                          acc.at[didx_v.at[j]],
                                          ssem.at[bp]).wait()

                pltpu.make_async_copy(table_hbm.at[sidx_v.at[j + 2]],
                                      rows_v.at[bp], gsem.at[bp]).start()

            return carry

        lax.fori_loop(0, n, body, 0)
        for j in range(max(0, n - NBUF), n):
            pltpu.make_async_copy(rows_v.at[j % NBUF],
                                  acc.at[didx_v.at[j]],
                                  ssem.at[j % NBUF]).wait()

    def grp(g, carry):
        emit_group(base + g * G, G)
        return carry

    lax.fori_loop(0, FG, grp, 0)

    @pl.when(wid < XTRA)
    def _tail():
        emit_group(base + FG * G, 1)

    plsc.subcore_barrier()
    pltpu.sync_copy(acc.at[pl.ds(s * RPS, RPS)],
                    out_hbm.at[c, pl.ds(s * RPS, RPS)])


_scatter = pl.kernel(
    _scatter_body,
    out_type=jax.ShapeDtypeStruct((NCORES, NROW, D), jnp.float32),
    mesh=_mesh,
    scratch_types=[
        pltpu.VMEM_SHARED((NROW, D), jnp.float32),
        pltpu.VMEM((NBUF, B, D), jnp.float32),
        pltpu.VMEM((G, B), jnp.int32),
        pltpu.VMEM((G, B), jnp.int32),
        pltpu.SemaphoreType.DMA((NBUF,)),
        pltpu.SemaphoreType.DMA((NBUF,)),
    ],
    compiler_params=_sc_params,
)


RB = 1024


def _dv_block(p_ref):
    i = pl.program_id(0)
    d = p_ref[0, pl.ds(i * RB, RB)] + p_ref[1, pl.ds(i * RB, RB)] + 1.0
    return lax.rsqrt(d)[:, None]


def _mm1_body(x_ref, w_ref, p_ref, o_ref):
    h = jnp.dot(x_ref[...], w_ref[...], preferred_element_type=jnp.float32)
    o_ref[...] = h * _dv_block(p_ref)


def _layer2_body(a0_ref, a1_ref, hp_ref, p_ref, b_ref, w_ref, o_ref):
    dv = _dv_block(p_ref)
    pre = (a0_ref[0] + a1_ref[0] + hp_ref[...]) * dv + b_ref[...][None, :]
    g = pre * 0.5 * (1.0 + lax.erf(pre * 0.7071067811865476))
    o_ref[...] = jnp.dot(g, w_ref[...],
                         preferred_element_type=jnp.float32) * dv


def _final_body(a0_ref, a1_ref, hp_ref, p_ref, b_ref, o_ref):
    o_ref[...] = ((a0_ref[0] + a1_ref[0] + hp_ref[...])
                  * _dv_block(p_ref) + b_ref[...][None, :])


_row_spec = pl.BlockSpec((RB, D), lambda i: (i, 0))
_a0_spec = pl.BlockSpec((1, RB, D), lambda i: (0, i, 0))
_a1_spec = pl.BlockSpec((1, RB, D), lambda i: (1, i, 0))
_p_spec = pl.BlockSpec((NCORES, NHIST), lambda i: (0, 0))
_w_spec = pl.BlockSpec((D, D), lambda i: (0, 0))
_b_spec = pl.BlockSpec((D,), lambda i: (0,))
_mat_out = jax.ShapeDtypeStruct((NPAD, D), jnp.float32)
_grid = (NPAD // RB,)

_mm1_call = pl.pallas_call(
    _mm1_body, grid=_grid,
    in_specs=[_row_spec, _w_spec, _p_spec],
    out_specs=_row_spec, out_shape=_mat_out)

_layer2_call = pl.pallas_call(
    _layer2_body, grid=_grid,
    in_specs=[_a0_spec, _a1_spec, _row_spec, _p_spec, _b_spec, _w_spec],
    out_specs=_row_spec, out_shape=_mat_out)

_final_call = pl.pallas_call(
    _final_body, grid=_grid,
    in_specs=[_a0_spec, _a1_spec, _row_spec, _p_spec, _b_spec],
    out_specs=_row_spec, out_shape=_mat_out)


@jax.jit
def kernel(x, edge_index, W1, b1, W2, b2):
    src2 = edge_index[0].astype(jnp.int32).reshape(CHUNKS, B)
    dst2 = edge_index[1].astype(jnp.int32).reshape(CHUNKS, B)
    x_p = jnp.zeros((NPAD, D), jnp.float32).at[:x.shape[0]].set(x)

    p = _hist(dst2)
    h1p = _mm1_call(x_p, W1, p)
    A1 = _scatter(h1p, src2, dst2)
    h2p = _layer2_call(A1, A1, h1p, p, b1, W2)
    A2 = _scatter(h2p, src2, dst2)
    return _final_call(A2, A2, h2p, p, b2)[:x.shape[0]]


# speedup vs baseline: 3.4066x; 1.0151x over previous
import jax
import jax.numpy as jnp
from jax import lax
from jax.experimental import pallas as pl
from jax.experimental.pallas import tpu as pltpu
from jax.experimental.pallas import tpu_sc as plsc

N_NODES = 10000
D = 128
B = 128
CHUNKS = 2500
NCORES = 2
NSUB = 16
NW = NCORES * NSUB
CPW = CHUNKS // NW
XTRA = CHUNKS - NW * CPW
NROW = 10016
RPS = NROW // NSUB
NHIST = 10240
RPSH = NHIST // NSUB
NBUF = 3
G = 6
FG = CPW // G
NPAD = 10240

_mesh = plsc.VectorSubcoreMesh(core_axis_name="c", subcore_axis_name="s")
_sc_params = pltpu.CompilerParams(use_tc_tiling_on_sc=False)


def _worker_base(c, s):
    wid = c * NSUB + s
    return wid, CPW * wid + jnp.minimum(wid, XTRA)


def _hist_body(dst_hbm, out_hbm, deg_acc, ones_v, idx_v, zero_v):
    c = lax.axis_index("c")
    s = lax.axis_index("s")
    wid, base = _worker_base(c, s)

    def fill_zero(i, carry):
        zero_v[pl.ds(i * 16, 16)] = jnp.zeros((16,), jnp.float32)
        return carry

    lax.fori_loop(0, RPSH // 16, fill_zero, 0)

    def fill_one(i, carry):
        ones_v[pl.ds(i * 16, 16)] = jnp.ones((16,), jnp.float32)
        return carry

    lax.fori_loop(0, B // 16, fill_one, 0)

    pltpu.sync_copy(zero_v, deg_acc.at[pl.ds(s * RPSH, RPSH)])
    plsc.subcore_barrier()

    def body(j, carry2):
        pltpu.sync_copy(ones_v, deg_acc.at[idx_v.at[j]], add=True)
        return carry2

    def grp(g, carry):
        pltpu.sync_copy(dst_hbm.at[pl.ds(base + g * G, G)], idx_v)
        lax.fori_loop(0, G, body, 0)
        return carry

    lax.fori_loop(0, FG, grp, 0)

    @pl.when(wid < XTRA)
    def _tail():
        pltpu.sync_copy(dst_hbm.at[pl.ds(base + FG * G, 1)],
                        idx_v.at[pl.ds(0, 1)])
        pltpu.sync_copy(ones_v, deg_acc.at[idx_v.at[0]], add=True)

    plsc.subcore_barrier()
    pltpu.sync_copy(deg_acc.at[pl.ds(s * RPSH, RPSH)],
                    out_hbm.at[c, pl.ds(s * RPSH, RPSH)])


_hist = pl.kernel(
    _hist_body,
    out_type=jax.ShapeDtypeStruct((NCORES, NHIST), jnp.float32),
    mesh=_mesh,
    scratch_types=[
        pltpu.VMEM_SHARED((NHIST,), jnp.float32),
        pltpu.VMEM((B,), jnp.float32),
        pltpu.VMEM((G, B), jnp.int32),
        pltpu.VMEM((RPSH,), jnp.float32),
    ],
    compiler_params=_sc_params,
)


def _scatter_body(table_hbm, src_hbm, dst_hbm, out_hbm,
                  acc, rows_v, sidx_v, didx_v, gsem, ssem):
    c = lax.axis_index("c")
    s = lax.axis_index("s")
    wid, base = _worker_base(c, s)

    def zero_row(r, carry):
        for k in range(D // 16):
            rows_v[0, r, pl.ds(k * 16, 16)] = jnp.zeros((16,), jnp.float32)
        return carry

    lax.fori_loop(0, B, zero_row, 0)
    for k in range(RPS // B):
        pltpu.sync_copy(rows_v.at[0], acc.at[pl.ds(s * RPS + k * B, B)])
    _rem = RPS % B
    pltpu.sync_copy(rows_v.at[0, pl.ds(0, _rem)],
                    acc.at[pl.ds(s * RPS + (RPS // B) * B, _rem)])
    plsc.subcore_barrier()

    def emit_group(gbase, n):
        pltpu.sync_copy(src_hbm.at[pl.ds(gbase, n)], sidx_v.at[pl.ds(0, n)])
        pltpu.sync_copy(dst_hbm.at[pl.ds(gbase, n)], didx_v.at[pl.ds(0, n)])
        for j in range(min(2, n)):
            pltpu.make_async_copy(table_hbm.at[sidx_v.at[j]],
                                  rows_v.at[j], gsem.at[j]).start()

        def body(j, carry):
            b = lax.rem(j, NBUF)
            pltpu.make_async_copy(table_hbm.at[sidx_v.at[j]],
                                  rows_v.at[b], gsem.at[b]).wait()
            pltpu.make_async_copy(rows_v.at[b], acc.at[didx_v.at[j]],
                                  ssem.at[b]).start(add=True)

            @pl.when(j + 2 < n)
            def _issue():
                bp = lax.rem(j + 2, NBUF)

                @pl.when(j >= 1)
                def _wait_prev():
                    pltpu.make_async_copy(rows_v.at[bp],
                                          acc.at[didx_v.at[j]],
                                          ssem.at[bp]).wait()

                pltpu.make_async_copy(table_hbm.at[sidx_v.at[j + 2]],
                                      rows_v.at[bp], gsem.at[bp]).start()

            return carry

        lax.fori_loop(0, n, body, 0)
        for j in range(max(0, n - NBUF), n):
            pltpu.make_async_copy(rows_v.at[j % NBUF],
                                  acc.at[didx_v.at[j]],
                                  ssem.at[j % NBUF]).wait()

    def grp(g, carry):
        emit_group(base + g * G, G)
        return carry

    lax.fori_loop(0, FG, grp, 0)

    @pl.when(wid < XTRA)
    def _tail():
        emit_group(base + FG * G, 1)

    plsc.subcore_barrier()
    pltpu.sync_copy(acc.at[pl.ds(s * RPS, RPS)],
                    out_hbm.at[c, pl.ds(s * RPS, RPS)])


_scatter = pl.kernel(
    _scatter_body,
    out_type=jax.ShapeDtypeStruct((NCORES, NROW, D), jnp.float32),
    mesh=_mesh,
    scratch_types=[
        pltpu.VMEM_SHARED((NROW, D), jnp.float32),
        pltpu.VMEM((NBUF, B, D), jnp.float32),
        pltpu.VMEM((G, B), jnp.int32),
        pltpu.VMEM((G, B), jnp.int32),
        pltpu.SemaphoreType.DMA((NBUF,)),
        pltpu.SemaphoreType.DMA((NBUF,)),
    ],
    compiler_params=_sc_params,
)


RB = 1024


def _dv_block(p_ref):
    i = pl.program_id(0)
    d = p_ref[0, pl.ds(i * RB, RB)] + p_ref[1, pl.ds(i * RB, RB)] + 1.0
    return lax.rsqrt(d)[:, None]


def _mm1_body(x_ref, w_ref, p_ref, o_ref):
    h = jnp.dot(x_ref[...], w_ref[...], preferred_element_type=jnp.float32)
    o_ref[...] = h * _dv_block(p_ref)


def _layer2_body(a0_ref, a1_ref, hp_ref, p_ref, b_ref, w_ref, o_ref):
    dv = _dv_block(p_ref)
    pre = (a0_ref[0] + a1_ref[0] + hp_ref[...]) * dv + b_ref[...][None, :]
    g = pre * 0.5 * (1.0 + lax.erf(pre * 0.7071067811865476))
    o_ref[...] = jnp.dot(g, w_ref[...],
                         preferred_element_type=jnp.float32) * dv


def _final_body(a0_ref, a1_ref, hp_ref, p_ref, b_ref, o_ref):
    o_ref[...] = ((a0_ref[0] + a1_ref[0] + hp_ref[...])
                  * _dv_block(p_ref) + b_ref[...][None, :])


_row_spec = pl.BlockSpec((RB, D), lambda i: (i, 0))
_a0_spec = pl.BlockSpec((1, RB, D), lambda i: (0, i, 0))
_a1_spec = pl.BlockSpec((1, RB, D), lambda i: (1, i, 0))
_p_spec = pl.BlockSpec((NCORES, NHIST), lambda i: (0, 0))
_w_spec = pl.BlockSpec((D, D), lambda i: (0, 0))
_b_spec = pl.BlockSpec((D,), lambda i: (0,))
_mat_out = jax.ShapeDtypeStruct((NPAD, D), jnp.float32)
_grid = (NPAD // RB,)

_mm1_call = pl.pallas_call(
    _mm1_body, grid=_grid,
    in_specs=[_row_spec, _w_spec, _p_spec],
    out_specs=_row_spec, out_shape=_mat_out)

_layer2_call = pl.pallas_call(
    _layer2_body, grid=_grid,
    in_specs=[_a0_spec, _a1_spec, _row_spec, _p_spec, _b_spec, _w_spec],
    out_specs=_row_spec, out_shape=_mat_out)

_final_call = pl.pallas_call(
    _final_body, grid=_grid,
    in_specs=[_a0_spec, _a1_spec, _row_spec, _p_spec, _b_spec],
    out_specs=_row_spec,
    out_shape=jax.ShapeDtypeStruct((N_NODES, D), jnp.float32))


@jax.jit
def kernel(x, edge_index, W1, b1, W2, b2):
    src2 = edge_index[0].astype(jnp.int32).reshape(CHUNKS, B)
    dst2 = edge_index[1].astype(jnp.int32).reshape(CHUNKS, B)
    x_p = jnp.zeros((NPAD, D), jnp.float32).at[:x.shape[0]].set(x)

    p = _hist(dst2)
    h1p = _mm1_call(x_p, W1, p)
    A1 = _scatter(h1p, src2, dst2)
    h2p = _layer2_call(A1, A1, h1p, p, b1, W2)
    A2 = _scatter(h2p, src2, dst2)
    return _final_call(A2, A2, h2p, p, b2)


# speedup vs baseline: 3.4673x; 1.0178x over previous
import jax
import jax.numpy as jnp
from jax import lax
from jax.experimental import pallas as pl
from jax.experimental.pallas import tpu as pltpu
from jax.experimental.pallas import tpu_sc as plsc

N_NODES = 10000
D = 128
B = 128
CHUNKS = 2500
NCORES = 2
NSUB = 16
NW = NCORES * NSUB
CPW = CHUNKS // NW
XTRA = CHUNKS - NW * CPW
NROW = 10016
RPS = NROW // NSUB
NHIST = 10240
RPSH = NHIST // NSUB
NBUF = 3
G = 6
FG = CPW // G
NPAD = 10240

_mesh = plsc.VectorSubcoreMesh(core_axis_name="c", subcore_axis_name="s")
_sc_params = pltpu.CompilerParams(use_tc_tiling_on_sc=False)


def _worker_base(c, s):
    wid = c * NSUB + s
    return wid, CPW * wid + jnp.minimum(wid, XTRA)


def _hist_body(dst_hbm, out_hbm, deg_acc, ones_v, idx_v, zero_v):
    c = lax.axis_index("c")
    s = lax.axis_index("s")
    wid, base = _worker_base(c, s)

    def fill_zero(i, carry):
        zero_v[pl.ds(i * 16, 16)] = jnp.zeros((16,), jnp.float32)
        return carry

    lax.fori_loop(0, RPSH // 16, fill_zero, 0)

    def fill_one(i, carry):
        ones_v[pl.ds(i * 16, 16)] = jnp.ones((16,), jnp.float32)
        return carry

    lax.fori_loop(0, B // 16, fill_one, 0)

    pltpu.sync_copy(zero_v, deg_acc.at[pl.ds(s * RPSH, RPSH)])
    plsc.subcore_barrier()

    def body(j, carry2):
        pltpu.sync_copy(ones_v, deg_acc.at[idx_v.at[j]], add=True)
        return carry2

    def grp(g, carry):
        pltpu.sync_copy(dst_hbm.at[pl.ds(base + g * G, G)], idx_v)
        lax.fori_loop(0, G, body, 0)
        return carry

    lax.fori_loop(0, FG, grp, 0)

    @pl.when(wid < XTRA)
    def _tail():
        pltpu.sync_copy(dst_hbm.at[pl.ds(base + FG * G, 1)],
                        idx_v.at[pl.ds(0, 1)])
        pltpu.sync_copy(ones_v, deg_acc.at[idx_v.at[0]], add=True)

    plsc.subcore_barrier()
    pltpu.sync_copy(deg_acc.at[pl.ds(s * RPSH, RPSH)],
                    out_hbm.at[c, pl.ds(s * RPSH, RPSH)])


_hist = pl.kernel(
    _hist_body,
    out_type=jax.ShapeDtypeStruct((NCORES, NHIST), jnp.float32),
    mesh=_mesh,
    scratch_types=[
        pltpu.VMEM_SHARED((NHIST,), jnp.float32),
        pltpu.VMEM((B,), jnp.float32),
        pltpu.VMEM((G, B), jnp.int32),
        pltpu.VMEM((RPSH,), jnp.float32),
    ],
    compiler_params=_sc_params,
)


def _scatter_body(table_hbm, src_hbm, dst_hbm, out_hbm,
                  acc, rows_v, sidx_v, didx_v, gsem, ssem):
    c = lax.axis_index("c")
    s = lax.axis_index("s")
    wid, base = _worker_base(c, s)

    def zero_row(r, carry):
        for k in range(D // 16):
            rows_v[0, r, pl.ds(k * 16, 16)] = jnp.zeros((16,), jnp.float32)
        return carry

    lax.fori_loop(0, B, zero_row, 0)
    for k in range(RPS // B):
        pltpu.sync_copy(rows_v.at[0], acc.at[pl.ds(s * RPS + k * B, B)])
    _rem = RPS % B
    pltpu.sync_copy(rows_v.at[0, pl.ds(0, _rem)],
                    acc.at[pl.ds(s * RPS + (RPS // B) * B, _rem)])
    plsc.subcore_barrier()

    def emit_group(gbase, n):
        pltpu.sync_copy(src_hbm.at[pl.ds(gbase, n)], sidx_v.at[pl.ds(0, n)])
        pltpu.sync_copy(dst_hbm.at[pl.ds(gbase, n)], didx_v.at[pl.ds(0, n)])
        for j in range(min(2, n)):
            pltpu.make_async_copy(table_hbm.at[sidx_v.at[j]],
                                  rows_v.at[j], gsem.at[j]).start()

        def body(j, carry):
            b = lax.rem(j, NBUF)
            pltpu.make_async_copy(table_hbm.at[sidx_v.at[j]],
                                  rows_v.at[b], gsem.at[b]).wait()
            pltpu.make_async_copy(rows_v.at[b], acc.at[didx_v.at[j]],
                                  ssem.at[b]).start(add=True)

            @pl.when(j + 2 < n)
            def _issue():
                bp = lax.rem(j + 2, NBUF)

                @pl.when(j >= 1)
                def _wait_prev():
                    pltpu.make_async_copy(rows_v.at[bp],
                                          acc.at[didx_v.at[j]],
                                          ssem.at[bp]).wait()

                pltpu.make_async_copy(table_hbm.at[sidx_v.at[j + 2]],
                                      rows_v.at[bp], gsem.at[bp]).start()

            return carry

        lax.fori_loop(0, n, body, 0)
        for j in range(max(0, n - NBUF), n):
            pltpu.make_async_copy(rows_v.at[j % NBUF],
                                  acc.at[didx_v.at[j]],
                                  ssem.at[j % NBUF]).wait()

    def grp(g, carry):
        emit_group(base + g * G, G)
        return carry

    lax.fori_loop(0, FG, grp, 0)

    @pl.when(wid < XTRA)
    def _tail():
        emit_group(base + FG * G, 1)

    plsc.subcore_barrier()
    pltpu.sync_copy(acc.at[pl.ds(s * RPS, RPS)],
                    out_hbm.at[c, pl.ds(s * RPS, RPS)])


_scatter = pl.kernel(
    _scatter_body,
    out_type=jax.ShapeDtypeStruct((NCORES, NROW, D), jnp.float32),
    mesh=_mesh,
    scratch_types=[
        pltpu.VMEM_SHARED((NROW, D), jnp.float32),
        pltpu.VMEM((NBUF, B, D), jnp.float32),
        pltpu.VMEM((G, B), jnp.int32),
        pltpu.VMEM((G, B), jnp.int32),
        pltpu.SemaphoreType.DMA((NBUF,)),
        pltpu.SemaphoreType.DMA((NBUF,)),
    ],
    compiler_params=_sc_params,
)


RB = 2048


def _dv_block(p_ref):
    i = pl.program_id(0)
    d = p_ref[0, pl.ds(i * RB, RB)] + p_ref[1, pl.ds(i * RB, RB)] + 1.0
    return lax.rsqrt(d)[:, None]


def _mm1_body(x_ref, w_ref, p_ref, o_ref):
    h = jnp.dot(x_ref[...], w_ref[...], preferred_element_type=jnp.float32)
    o_ref[...] = h * _dv_block(p_ref)


def _layer2_body(a0_ref, a1_ref, hp_ref, p_ref, b_ref, w_ref, o_ref):
    dv = _dv_block(p_ref)
    pre = (a0_ref[0] + a1_ref[0] + hp_ref[...]) * dv + b_ref[...][None, :]
    g = pre * 0.5 * (1.0 + lax.erf(pre * 0.7071067811865476))
    o_ref[...] = jnp.dot(g, w_ref[...],
                         preferred_element_type=jnp.float32) * dv


def _final_body(a0_ref, a1_ref, hp_ref, p_ref, b_ref, o_ref):
    o_ref[...] = ((a0_ref[0] + a1_ref[0] + hp_ref[...])
                  * _dv_block(p_ref) + b_ref[...][None, :])


_row_spec = pl.BlockSpec((RB, D), lambda i: (i, 0))
_a0_spec = pl.BlockSpec((1, RB, D), lambda i: (0, i, 0))
_a1_spec = pl.BlockSpec((1, RB, D), lambda i: (1, i, 0))
_p_spec = pl.BlockSpec((NCORES, NHIST), lambda i: (0, 0))
_w_spec = pl.BlockSpec((D, D), lambda i: (0, 0))
_b_spec = pl.BlockSpec((D,), lambda i: (0,))
_mat_out = jax.ShapeDtypeStruct((NPAD, D), jnp.float32)
_grid = (NPAD // RB,)

_mm1_call = pl.pallas_call(
    _mm1_body, grid=_grid,
    in_specs=[_row_spec, _w_spec, _p_spec],
    out_specs=_row_spec, out_shape=_mat_out)

_layer2_call = pl.pallas_call(
    _layer2_body, grid=_grid,
    in_specs=[_a0_spec, _a1_spec, _row_spec, _p_spec, _b_spec, _w_spec],
    out_specs=_row_spec, out_shape=_mat_out)

_final_call = pl.pallas_call(
    _final_body, grid=_grid,
    in_specs=[_a0_spec, _a1_spec, _row_spec, _p_spec, _b_spec],
    out_specs=_row_spec,
    out_shape=jax.ShapeDtypeStruct((N_NODES, D), jnp.float32))


@jax.jit
def kernel(x, edge_index, W1, b1, W2, b2):
    src2 = edge_index[0].astype(jnp.int32).reshape(CHUNKS, B)
    dst2 = edge_index[1].astype(jnp.int32).reshape(CHUNKS, B)
    x_p = jnp.zeros((NPAD, D), jnp.float32).at[:x.shape[0]].set(x)

    p = _hist(dst2)
    h1p = _mm1_call(x_p, W1, p)
    A1 = _scatter(h1p, src2, dst2)
    h2p = _layer2_call(A1, A1, h1p, p, b1, W2)
    A2 = _scatter(h2p, src2, dst2)
    return _final_call(A2, A2, h2p, p, b2)
